# baseline (speedup 1.0000x reference)
import functools

import jax
import jax.numpy as jnp
from jax import lax
from jax.experimental import pallas as pl
from jax.experimental.pallas import tpu as pltpu
from jax.experimental.pallas import tpu_sc as plsc

B = 16384
D = 16
L = 16
NW = 32
NC = 2
T_TILES = 7813
WINS = 31
CAP = 32
STG = 128
OUTROWS = B + 8

_mesh = plsc.VectorSubcoreMesh(core_axis_name="c", subcore_axis_name="s")


@functools.partial(
    pl.kernel,
    mesh=_mesh,
    compiler_params=pltpu.CompilerParams(needs_layout_passes=False),
    out_type=(jax.ShapeDtypeStruct((OUTROWS, 128), jnp.float32),
              jax.ShapeDtypeStruct((OUTROWS, 128), jnp.float32)),
    scratch_types=[
        pltpu.VMEM((B,), jnp.int32),
        pltpu.VMEM((2, D, 8 * 128), jnp.float32),
        pltpu.VMEM((WINS * 16 * CAP,), jnp.int32),
        pltpu.VMEM((WINS * 16 * CAP,), jnp.int32),
        pltpu.VMEM((WINS * 16,), jnp.int32),
        pltpu.VMEM((L, 128), jnp.float32),
        pltpu.VMEM((L, 128), jnp.float32),
        pltpu.VMEM((L,), jnp.int32),
        pltpu.VMEM((L,), jnp.int32),
        pltpu.SemaphoreType.DMA,
        pltpu.SemaphoreType.DMA,
        pltpu.SemaphoreType.DMA,
        pltpu.SemaphoreType.DMA,
    ],
)
def _extract(user_hbm, song_hbm, uwt_hbm, swt_hbm, u_out, s_out,
             idxl, winbuf, buckets, bucketc, hist, stga, stgb, stia, stib,
             sem_w0, sem_w1, sem_f0, sem_f1):
    w = lax.axis_index("s") * NC + lax.axis_index("c")
    lanev = lax.iota(jnp.int32, L)
    zero16 = jnp.zeros((L,), jnp.int32)
    one16 = jnp.ones((L,), jnp.int32)
    dumvec = zero16 + (B + (w & 7))

    sems_w = (sem_w0, sem_w1)
    sems_f = (sem_f0, sem_f1)
    stgs = (stga, stgb)
    stis = (stia, stib)

    def do_table(idx_hbm, tbl_hbm, out_hbm):
        def fire_window(win, p):
            for j in range(8):
                t_j = jnp.minimum(win * 256 + j * 32 + w, T_TILES - 1)
                off = pl.multiple_of(t_j * 128, 128)
                pltpu.async_copy(tbl_hbm.at[:, pl.ds(off, 128)],
                                 winbuf.at[p].at[:, pl.ds(j * 128, 128)],
                                 sems_w[p])

        def drain_window(p):
            pltpu.make_async_copy(tbl_hbm.at[:, pl.ds(0, 8 * 128)],
                                  winbuf.at[p], sems_w[p]).wait()

        def drain_stg(q):
            pltpu.make_async_copy(tbl_hbm.at[:, pl.ds(0, 128)],
                                  stgs[q], sems_f[q]).wait()

        fire_window(0, 0)
        for q in range(2):
            pltpu.async_copy(tbl_hbm.at[:, pl.ds(0, 128)], stgs[q], sems_f[q])
        pltpu.sync_copy(idx_hbm, idxl)
        for m in range(WINS):
            hist[pl.ds(m * L, L)] = zero16

        def binbody(t, carry):
            v = idxl[pl.ds(t * L, L)]
            mine = (lax.shift_right_logical(v, 7) & 31) == w
            cell = lax.shift_right_logical(v, 15) * L + lanev
            cnt = plsc.load_gather(hist, [cell])
            ok = mine & (cnt < CAP)
            slot = cell * CAP + cnt
            plsc.store_scatter(buckets, [slot], t * L + lanev, mask=ok)
            plsc.store_scatter(bucketc, [slot], v, mask=ok)
            plsc.addupdate_scatter(hist, [cell], one16, mask=mine)
            return carry

        lax.fori_loop(0, B // L, binbody, 0, unroll=4)

        def winstep(win, p):
            fire_window(jnp.minimum(win + 1, WINS - 1), 1 - p)
            drain_window(p)
            wbuf = winbuf.at[p]
            hvec = jnp.minimum(hist[pl.ds(win * L, L)], CAP)
            maxc = jnp.max(hvec)
            cellbase = (win * L + lanev) * CAP

            def sstep(s, q):
                valid = hvec > s
                bvec = plsc.load_gather(buckets, [cellbase + s])
                cvec = plsc.load_gather(bucketc, [cellbase + s])
                jv = (lax.shift_right_logical(
                    lax.shift_right_logical(cvec, 7) - w, 5)) & 7
                col = jv * 128 + (cvec & 127)
                drain_stg(q)
                for d in range(D):
                    vals = plsc.load_gather(wbuf, [zero16 + d, col])
                    plsc.store_scatter(stgs[q], [lanev, zero16 + d], vals)
                stis[q][pl.ds(0, L)] = jnp.where(valid, bvec, dumvec)
                pltpu.async_copy(stgs[q], out_hbm.at[stis[q]], sems_f[q])

            def spair(i, carry):
                sstep(2 * i, 0)

                @pl.when(2 * i + 1 < maxc)
                def _odd():
                    sstep(2 * i + 1, 1)

                return carry

            lax.fori_loop(0, (maxc + 1) // 2, spair, 0)

        def winpair(i, carry):
            winstep(2 * i, 0)
            winstep(2 * i + 1, 1)
            return carry

        lax.fori_loop(0, WINS // 2, winpair, 0)
        winstep(WINS - 1, 0)
        drain_window(1)
        drain_stg(0)
        drain_stg(1)

    do_table(user_hbm, uwt_hbm, u_out)
    do_table(song_hbm, swt_hbm, s_out)


BPW = B // NW
CHUNK = 128


@functools.partial(
    pl.kernel,
    mesh=_mesh,
    compiler_params=pltpu.CompilerParams(needs_layout_passes=False),
    out_type=jax.ShapeDtypeStruct((B,), jnp.float32),
    scratch_types=[
        pltpu.VMEM((2, CHUNK, 128), jnp.float32),
        pltpu.VMEM((2, CHUNK, 128), jnp.float32),
        pltpu.VMEM((BPW,), jnp.float32),
        pltpu.SemaphoreType.DMA,
        pltpu.SemaphoreType.DMA,
    ],
)
def _dot(u_hbm, s_hbm, out_hbm, ubuf, sbuf, out_v, sem_u, sem_s):
    wid = lax.axis_index("s") * NC + lax.axis_index("c")
    base = wid * BPW
    lane = lax.iota(jnp.int32, L)

    def start(jc):
        p = jc % 2
        cu = pltpu.async_copy(u_hbm.at[pl.ds(base + jc * CHUNK, CHUNK)],
                              ubuf.at[p], sem_u)
        cs = pltpu.async_copy(s_hbm.at[pl.ds(base + jc * CHUNK, CHUNK)],
                              sbuf.at[p], sem_s)
        return cu, cs

    pend = start(0)
    for jc in range(BPW // CHUNK):
        cu, cs = pend
        cu.wait()
        cs.wait()
        if jc + 1 < BPW // CHUNK:
            pend = start(jc + 1)
        p = jc % 2
        for g in range(CHUNK // L):
            rows = g * L + lane
            acc = jnp.zeros((L,), jnp.float32)
            for d in range(D):
                cold = jnp.full((L,), d, jnp.int32)
                acc = acc + (plsc.load_gather(ubuf.at[p], [rows, cold]) *
                             plsc.load_gather(sbuf.at[p], [rows, cold]))
            out_v[pl.ds(jc * CHUNK + g * L, L)] = jnp.maximum(acc, 0.0)

    pltpu.sync_copy(out_v, out_hbm.at[pl.ds(base, BPW)])


def kernel(user, song, user_weight, song_weight):
    u_rows, s_rows = _extract(user, song, user_weight.T, song_weight.T)
    return _dot(u_rows, s_rows)


# speedup vs baseline: 2.9788x; 2.9788x over previous
import functools

import jax
import jax.numpy as jnp
from jax import lax
from jax.experimental import pallas as pl
from jax.experimental.pallas import tpu as pltpu
from jax.experimental.pallas import tpu_sc as plsc

B = 16384
D = 16
L = 16
NW = 32
NC = 2
T_TILES = 7813
WINS = 31
CAP = 32
STG = 128
OUTROWS = B + 8

_mesh = plsc.VectorSubcoreMesh(core_axis_name="c", subcore_axis_name="s")


@functools.partial(
    pl.kernel,
    mesh=_mesh,
    compiler_params=pltpu.CompilerParams(needs_layout_passes=False),
    out_type=(jax.ShapeDtypeStruct((OUTROWS, 128), jnp.float32),
              jax.ShapeDtypeStruct((OUTROWS, 128), jnp.float32)),
    scratch_types=[
        pltpu.VMEM((B,), jnp.int32),
        pltpu.VMEM((2, D, 8 * 128), jnp.float32),
        pltpu.VMEM((WINS * 16 * CAP,), jnp.int32),
        pltpu.VMEM((WINS * 16 * CAP,), jnp.int32),
        pltpu.VMEM((WINS * 16,), jnp.int32),
        pltpu.VMEM((STG, 128), jnp.float32),
        pltpu.VMEM((STG,), jnp.int32),
        pltpu.SemaphoreType.DMA,
        pltpu.SemaphoreType.DMA,
        pltpu.SemaphoreType.DMA,
    ],
)
def _extract(user_hbm, song_hbm, uwt_hbm, swt_hbm, u_out, s_out,
             idxl, winbuf, buckets, bucketc, hist, stg, stidx,
             sem_w0, sem_w1, sem_f):
    w = lax.axis_index("s") * NC + lax.axis_index("c")
    lanev = lax.iota(jnp.int32, L)
    zero16 = jnp.zeros((L,), jnp.int32)
    one16 = jnp.ones((L,), jnp.int32)
    dumvec = zero16 + (B + (w & 7))

    sems_w = (sem_w0, sem_w1)

    def reset_stidx():
        for m in range(STG // L):
            stidx[pl.ds(m * L, L)] = dumvec

    def do_table(idx_hbm, tbl_hbm, out_hbm):
        def fire_window(win, p):
            for j in range(8):
                t_j = jnp.minimum(win * 256 + j * 32 + w, T_TILES - 1)
                off = pl.multiple_of(t_j * 128, 128)
                pltpu.async_copy(tbl_hbm.at[:, pl.ds(off, 128)],
                                 winbuf.at[p].at[:, pl.ds(j * 128, 128)],
                                 sems_w[p])

        def drain_window(p):
            pltpu.make_async_copy(tbl_hbm.at[:, pl.ds(0, 8 * 128)],
                                  winbuf.at[p], sems_w[p]).wait()

        def flush():
            pltpu.async_copy(stg, out_hbm.at[stidx], sem_f).wait()
            reset_stidx()

        fire_window(0, 0)
        pltpu.sync_copy(idx_hbm, idxl)
        for m in range(WINS):
            hist[pl.ds(m * L, L)] = zero16
        reset_stidx()

        def binbody(t, carry):
            v = idxl[pl.ds(t * L, L)]
            mine = (lax.shift_right_logical(v, 7) & 31) == w
            cell = lax.shift_right_logical(v, 15) * L + lanev
            cnt = plsc.load_gather(hist, [cell])
            ok = mine & (cnt < CAP)
            slot = cell * CAP + cnt
            plsc.store_scatter(buckets, [slot], t * L + lanev, mask=ok)
            plsc.store_scatter(bucketc, [slot], v, mask=ok)
            plsc.addupdate_scatter(hist, [cell], one16, mask=mine)
            return carry

        lax.fori_loop(0, B // L, binbody, 0, unroll=4)

        def winstep(win, p, h):
            fire_window(jnp.minimum(win + 1, WINS - 1), 1 - p)
            drain_window(p)
            wbuf = winbuf.at[p]
            hvec = jnp.minimum(hist[pl.ds(win * L, L)], CAP)
            maxc = jnp.max(hvec)
            cellbase = (win * L + lanev) * CAP

            def sstep(s, h):
                valid = hvec > s
                vi = valid.astype(jnp.int32)
                cnt_s = jnp.sum(vi)
                bvec = plsc.load_gather(buckets, [cellbase + s])
                cvec = plsc.load_gather(bucketc, [cellbase + s])
                jv = (lax.shift_right_logical(
                    lax.shift_right_logical(cvec, 7) - w, 5)) & 7
                col = jv * 128 + (cvec & 127)

                @pl.when(h > STG - L)
                def _full():
                    flush()

                h2 = jnp.where(h > STG - L, 0, h)
                rows_t = h2 + plsc.cumsum(vi) - 1
                for d in range(D):
                    vals = plsc.load_gather(wbuf, [zero16 + d, col])
                    plsc.store_scatter(stg, [rows_t, zero16 + d], vals,
                                       mask=valid)
                plsc.store_scatter(stidx, [rows_t], bvec, mask=valid)
                return h2 + cnt_s

            return lax.fori_loop(0, maxc, sstep, h)

        def winpair(i, h):
            h = winstep(2 * i, 0, h)
            h = winstep(2 * i + 1, 1, h)
            return h

        h = lax.fori_loop(0, WINS // 2, winpair, 0)
        winstep(WINS - 1, 0, h)
        drain_window(1)
        flush()

    do_table(user_hbm, uwt_hbm, u_out)
    do_table(song_hbm, swt_hbm, s_out)


BPW = B // NW
CHUNK = 128


@functools.partial(
    pl.kernel,
    mesh=_mesh,
    compiler_params=pltpu.CompilerParams(needs_layout_passes=False),
    out_type=jax.ShapeDtypeStruct((B,), jnp.float32),
    scratch_types=[
        pltpu.VMEM((2, CHUNK, 128), jnp.float32),
        pltpu.VMEM((2, CHUNK, 128), jnp.float32),
        pltpu.VMEM((BPW,), jnp.float32),
        pltpu.SemaphoreType.DMA,
        pltpu.SemaphoreType.DMA,
    ],
)
def _dot(u_hbm, s_hbm, out_hbm, ubuf, sbuf, out_v, sem_u, sem_s):
    wid = lax.axis_index("s") * NC + lax.axis_index("c")
    base = wid * BPW
    lane = lax.iota(jnp.int32, L)

    def start(jc):
        p = jc % 2
        cu = pltpu.async_copy(u_hbm.at[pl.ds(base + jc * CHUNK, CHUNK)],
                              ubuf.at[p], sem_u)
        cs = pltpu.async_copy(s_hbm.at[pl.ds(base + jc * CHUNK, CHUNK)],
                              sbuf.at[p], sem_s)
        return cu, cs

    pend = start(0)
    for jc in range(BPW // CHUNK):
        cu, cs = pend
        cu.wait()
        cs.wait()
        if jc + 1 < BPW // CHUNK:
            pend = start(jc + 1)
        p = jc % 2
        for g in range(CHUNK // L):
            rows = g * L + lane
            acc = jnp.zeros((L,), jnp.float32)
            for d in range(D):
                cold = jnp.full((L,), d, jnp.int32)
                acc = acc + (plsc.load_gather(ubuf.at[p], [rows, cold]) *
                             plsc.load_gather(sbuf.at[p], [rows, cold]))
            out_v[pl.ds(jc * CHUNK + g * L, L)] = jnp.maximum(acc, 0.0)

    pltpu.sync_copy(out_v, out_hbm.at[pl.ds(base, BPW)])


def kernel(user, song, user_weight, song_weight):
    u_rows, s_rows = _extract(user, song, user_weight.T, song_weight.T)
    return _dot(u_rows, s_rows)
